# baseline (speedup 1.0000x reference)
import jax
import jax.numpy as jnp
from jax.experimental import pallas as pl
from jax.experimental.pallas import tpu as pltpu

_B = 32
_H = 198
_S0 = _H * _H


def _conv_step(src, dst, w_ref, sb_ref, stride, m, cin, cout, tm=512):
    scale = sb_ref[0:1, :]
    bias = sb_ref[1:2, :]
    for c0 in range(0, m, tm):
        sz = min(tm, m - c0)
        acc = None
        for dy in range(3):
            for dx in range(3):
                t = dy * 3 + dx
                off = c0 + dy * stride + dx
                a = src[off:off + sz, 0:cin]
                p = jnp.dot(a, w_ref[t], preferred_element_type=jnp.float32)
                acc = p if acc is None else acc + p
        dst[c0:c0 + sz, 0:cout] = jnp.maximum(
            acc * scale + bias, 0.0).astype(jnp.bfloat16)


def _pool_step(src, dst, k, stride, n_rows, n_cols, c):
    span = k * n_cols
    for r in range(n_rows):
        m1 = None
        for i in range(k):
            base = (k * r + i) * stride
            sl = src[base:base + span, 0:c]
            m1 = sl if m1 is None else jnp.maximum(m1, sl)
        if k == 2:
            w32 = pltpu.bitcast(m1, jnp.int32)
            lo = pltpu.bitcast(jnp.left_shift(w32, 16), jnp.float32)
            hi = pltpu.bitcast(jnp.bitwise_and(w32, jnp.int32(-65536)),
                               jnp.float32)
            red = jnp.maximum(lo, hi).astype(jnp.bfloat16)
        else:
            red = jnp.max(m1.reshape(n_cols, k, c), axis=1)
        dst[r * n_cols:(r + 1) * n_cols, 0:c] = red


def _k1_kernel(x_ref, w0, w1, w2, s0, s1, s2, out_ref, sa, sb, sc):
    _conv_step(x_ref.at[0, 0], sa, w0, s0, 198, 19798, 12, 128, tm=1024)
    _conv_step(sa, sb, w1, s1, 198, 19400, 128, 128, tm=1024)
    _conv_step(sb, sc, w2, s2, 198, 19002, 128, 256)
    _pool_step(sc, out_ref.at[0, 0], 2, 198, 48, 96, 256)


def _k2_kernel(x_ref, w3, w4, w5, s3, s4, s5, out_ref, sa, sb, sc):
    _conv_step(x_ref.at[0], sa, w3, s3, 96, 9022, 128, 256)
    _conv_step(sa, sb, w4, s4, 96, 8828, 256, 256)
    _conv_step(sb, sc, w5, s5, 96, 8634, 256, 256)
    _pool_step(sc, out_ref.at[0], 3, 96, 30, 30, 256)


def _k3_kernel(x_ref, w6, w7, w8, w9, w10, s6, s7, s8, s9, s10,
               w1_ref, b1_ref, w2_ref, b2_ref, out_ref, sa, sb, sc):
    _conv_step(x_ref.at[0], sa, w6, s6, 30, 838, 128, 256)
    _conv_step(sa, sb, w7, s7, 30, 776, 256, 256)
    _conv_step(sb, sa, w8, s8, 30, 714, 256, 256)
    _pool_step(sa, sb, 3, 30, 8, 8, 256)
    _conv_step(sb, sa, w9, s9, 8, 46, 256, 256)
    _conv_step(sa, sc, w10, s10, 8, 28, 256, 256)
    s = (sc[0:4, :].astype(jnp.float32)
         + sc[8:12, :].astype(jnp.float32)
         + sc[16:20, :].astype(jnp.float32)
         + sc[24:28, :].astype(jnp.float32))
    xp = jnp.sum(s, axis=0, keepdims=True) * (1.0 / 16.0)
    h = jnp.dot(xp, w1_ref[...], preferred_element_type=jnp.float32) + b1_ref[...]
    h = jnp.maximum(h, 0.0)
    out_ref[0] = jnp.dot(h, w2_ref[...], preferred_element_type=jnp.float32) + b2_ref[...]


def _gw(w, g):
    k9 = w.reshape(9, w.shape[2], w.shape[3])
    if g == 1:
        return k9
    eye = jnp.eye(g, dtype=w.dtype)
    bd = jnp.einsum('gh,tij->tgihj', eye, k9)
    return bd.reshape(9, g * w.shape[2], g * w.shape[3])


def _const_specs(arrs):
    c2 = lambda i, j: (0, 0)
    c3 = lambda i, j: (0, 0, 0)
    return [pl.BlockSpec(a.shape, c3 if a.ndim == 3 else c2) for a in arrs]


def kernel(x, conv0_w, conv0_sb, conv1_w, conv1_sb, conv2_w, conv2_sb,
           conv3_w, conv3_sb, conv4_w, conv4_sb, conv5_w, conv5_sb,
           conv6_w, conv6_sb, conv7_w, conv7_sb, conv8_w, conv8_sb,
           conv9_w, conv9_sb, conv10_w, conv10_sb, w1, b1, w2, b2):
    f32 = jnp.float32

    xb = x.astype(jnp.bfloat16).reshape(8, 4, _S0, 3)
    xg = xb.transpose(0, 2, 1, 3).reshape(8, _S0, 12)
    xs = jnp.stack([xg[:, 198 * 96 * t: 198 * (96 * t + 102), :]
                    for t in range(2)], axis=1)

    k1_w = [_gw(conv0_w, 4), _gw(conv1_w, 4), _gw(conv2_w, 4)]
    k1_sb = [jnp.tile(s, (1, 4)) for s in (conv0_sb, conv1_sb, conv2_sb)]

    b1p = pl.pallas_call(
        _k1_kernel,
        out_shape=jax.ShapeDtypeStruct((8, 2, 4608, 256), jnp.bfloat16),
        grid=(8, 2),
        in_specs=[pl.BlockSpec((1, 1, 20196, 12), lambda i, j: (i, j, 0, 0))]
        + _const_specs(k1_w + k1_sb),
        out_specs=pl.BlockSpec((1, 1, 4608, 256), lambda i, j: (i, j, 0, 0)),
        scratch_shapes=[pltpu.VMEM((19808, 128), jnp.bfloat16),
                        pltpu.VMEM((19408, 128), jnp.bfloat16),
                        pltpu.VMEM((19008, 256), jnp.bfloat16)],
        compiler_params=pltpu.CompilerParams(
            dimension_semantics=("parallel", "parallel"),
            vmem_limit_bytes=56 * 1024 * 1024),
    )(xs, *k1_w, *k1_sb)

    b1r = b1p.reshape(8, 9216, 2, 128).transpose(0, 2, 1, 3).reshape(16, 9216, 128)

    k2_w = [_gw(conv3_w, 2), _gw(conv4_w, 2), _gw(conv5_w, 2)]
    k2_sb = [jnp.tile(s, (1, 2)) for s in (conv3_sb, conv4_sb, conv5_sb)]
    c2g = lambda i: (0, 0)
    c3g = lambda i: (0, 0, 0)

    b2p = pl.pallas_call(
        _k2_kernel,
        out_shape=jax.ShapeDtypeStruct((16, 900, 256), jnp.bfloat16),
        grid=(16,),
        in_specs=[pl.BlockSpec((1, 9216, 128), lambda i: (i, 0, 0))]
        + [pl.BlockSpec(a.shape, c3g) for a in k2_w]
        + [pl.BlockSpec(a.shape, c2g) for a in k2_sb],
        out_specs=pl.BlockSpec((1, 900, 256), lambda i: (i, 0, 0)),
        scratch_shapes=[pltpu.VMEM((9024, 256), jnp.bfloat16),
                        pltpu.VMEM((8832, 256), jnp.bfloat16),
                        pltpu.VMEM((8640, 256), jnp.bfloat16)],
        compiler_params=pltpu.CompilerParams(
            dimension_semantics=("parallel",),
            vmem_limit_bytes=40 * 1024 * 1024),
    )(b1r, *k2_w, *k2_sb)

    b2r = b2p.reshape(16, 900, 2, 128).transpose(0, 2, 1, 3).reshape(32, 900, 128)

    k3_w = [_gw(w, 1) for w in (conv6_w, conv7_w, conv8_w, conv9_w, conv10_w)]
    k3_sb = [conv6_sb, conv7_sb, conv8_sb, conv9_sb, conv10_sb]
    c2 = lambda i: (0, 0)
    c3 = lambda i: (0, 0, 0)

    out = pl.pallas_call(
        _k3_kernel,
        out_shape=jax.ShapeDtypeStruct((_B, 1, 20), f32),
        grid=(_B,),
        in_specs=[pl.BlockSpec((1, 900, 128), lambda i: (i, 0, 0))]
        + [pl.BlockSpec(a.shape, c3) for a in k3_w]
        + [pl.BlockSpec(a.shape, c2) for a in k3_sb]
        + [pl.BlockSpec(w1.shape, c2), pl.BlockSpec((1, 100), c2),
           pl.BlockSpec(w2.shape, c2), pl.BlockSpec((1, 20), c2)],
        out_specs=pl.BlockSpec((1, 1, 20), lambda i: (i, 0, 0)),
        scratch_shapes=[pltpu.VMEM((848, 256), jnp.bfloat16),
                        pltpu.VMEM((784, 256), jnp.bfloat16),
                        pltpu.VMEM((32, 256), jnp.bfloat16)],
        compiler_params=pltpu.CompilerParams(
            dimension_semantics=("parallel",),
            vmem_limit_bytes=40 * 1024 * 1024),
    )(b2r, *k3_w, *k3_sb, w1, b1.reshape(1, 100), w2, b2.reshape(1, 20))
    return out.reshape(_B, 20)


# speedup vs baseline: 1.0986x; 1.0986x over previous
import jax
import jax.numpy as jnp
from jax.experimental import pallas as pl
from jax.experimental.pallas import tpu as pltpu

_B = 32
_H = 198
_S0 = _H * _H


def _conv_step(src, dst, w_ref, sb_ref, stride, m, cin, cout, tm=512):
    scale = sb_ref[0:1, :]
    bias = sb_ref[1:2, :]
    for c0 in range(0, m, tm):
        sz = min(tm, m - c0)
        acc = None
        for dy in range(3):
            for dx in range(3):
                t = dy * 3 + dx
                off = c0 + dy * stride + dx
                a = src[off:off + sz, 0:cin]
                p = jnp.dot(a, w_ref[t], preferred_element_type=jnp.float32)
                acc = p if acc is None else acc + p
        dst[c0:c0 + sz, 0:cout] = jnp.maximum(
            acc * scale + bias, 0.0).astype(jnp.bfloat16)


def _pool_step(src, dst, k, stride, n_rows, n_cols, c):
    span = k * n_cols
    for r in range(n_rows):
        m1 = None
        for i in range(k):
            base = (k * r + i) * stride
            sl = src[base:base + span, 0:c]
            m1 = sl if m1 is None else jnp.maximum(m1, sl)
        if k == 2:
            w32 = pltpu.bitcast(m1, jnp.int32)
            lo = pltpu.bitcast(jnp.left_shift(w32, 16), jnp.float32)
            hi = pltpu.bitcast(jnp.bitwise_and(w32, jnp.int32(-65536)),
                               jnp.float32)
            red = jnp.maximum(lo, hi).astype(jnp.bfloat16)
        else:
            red = jnp.max(m1.reshape(n_cols, k, c), axis=1)
        dst[r * n_cols:(r + 1) * n_cols, 0:c] = red


def _k1_kernel(x_ref, w0, w1, w2, s0, s1, s2, out_ref, sa, sb, sc):
    _conv_step(x_ref.at[0, 0], sa, w0, s0, 198, 10294, 12, 128, tm=1024)
    _conv_step(sa, sb, w1, s1, 198, 9896, 128, 128, tm=1024)
    _conv_step(sb, sc, w2, s2, 198, 9498, 128, 256)
    _pool_step(sc, out_ref.at[0, 0], 2, 198, 24, 96, 256)


def _k2_kernel(x_ref, w3, w4, w5, s3, s4, s5, out_ref, sa, sb, sc):
    _conv_step(x_ref.at[0], sa, w3, s3, 96, 9022, 128, 256)
    _conv_step(sa, sb, w4, s4, 96, 8828, 256, 256)
    _conv_step(sb, sc, w5, s5, 96, 8634, 256, 256)
    _pool_step(sc, out_ref.at[0], 3, 96, 30, 30, 256)


def _k3_kernel(x_ref, w6, w7, w8, w9, w10, s6, s7, s8, s9, s10,
               w1_ref, b1_ref, w2_ref, b2_ref, out_ref, sa, sb, sc):
    _conv_step(x_ref.at[0], sa, w6, s6, 30, 838, 128, 256)
    _conv_step(sa, sb, w7, s7, 30, 776, 256, 256)
    _conv_step(sb, sa, w8, s8, 30, 714, 256, 256)
    _pool_step(sa, sb, 3, 30, 8, 8, 256)
    _conv_step(sb, sa, w9, s9, 8, 46, 256, 256)
    _conv_step(sa, sc, w10, s10, 8, 28, 256, 256)
    s = (sc[0:4, :].astype(jnp.float32)
         + sc[8:12, :].astype(jnp.float32)
         + sc[16:20, :].astype(jnp.float32)
         + sc[24:28, :].astype(jnp.float32))
    xp = jnp.sum(s, axis=0, keepdims=True) * (1.0 / 16.0)
    h = jnp.dot(xp, w1_ref[...], preferred_element_type=jnp.float32) + b1_ref[...]
    h = jnp.maximum(h, 0.0)
    out_ref[0] = jnp.dot(h, w2_ref[...], preferred_element_type=jnp.float32) + b2_ref[...]


def _gw(w, g):
    k9 = w.reshape(9, w.shape[2], w.shape[3])
    if g == 1:
        return k9
    eye = jnp.eye(g, dtype=w.dtype)
    bd = jnp.einsum('gh,tij->tgihj', eye, k9)
    return bd.reshape(9, g * w.shape[2], g * w.shape[3])


def _const_specs(arrs):
    c2 = lambda i, j: (0, 0)
    c3 = lambda i, j: (0, 0, 0)
    return [pl.BlockSpec(a.shape, c3 if a.ndim == 3 else c2) for a in arrs]


def kernel(x, conv0_w, conv0_sb, conv1_w, conv1_sb, conv2_w, conv2_sb,
           conv3_w, conv3_sb, conv4_w, conv4_sb, conv5_w, conv5_sb,
           conv6_w, conv6_sb, conv7_w, conv7_sb, conv8_w, conv8_sb,
           conv9_w, conv9_sb, conv10_w, conv10_sb, w1, b1, w2, b2):
    f32 = jnp.float32

    xb = x.astype(jnp.bfloat16).reshape(8, 4, _S0, 3)
    xg = xb.transpose(0, 2, 1, 3).reshape(8, _S0, 12)
    xs = jnp.stack([xg[:, 198 * 48 * t: 198 * (48 * t + 54), :]
                    for t in range(4)], axis=1)

    k1_w = [_gw(conv0_w, 4), _gw(conv1_w, 4), _gw(conv2_w, 4)]
    k1_sb = [jnp.tile(s, (1, 4)) for s in (conv0_sb, conv1_sb, conv2_sb)]

    b1p = pl.pallas_call(
        _k1_kernel,
        out_shape=jax.ShapeDtypeStruct((8, 4, 2304, 256), jnp.bfloat16),
        grid=(8, 4),
        in_specs=[pl.BlockSpec((1, 1, 10692, 12), lambda i, j: (i, j, 0, 0))]
        + _const_specs(k1_w + k1_sb),
        out_specs=pl.BlockSpec((1, 1, 2304, 256), lambda i, j: (i, j, 0, 0)),
        scratch_shapes=[pltpu.VMEM((10304, 128), jnp.bfloat16),
                        pltpu.VMEM((9904, 128), jnp.bfloat16),
                        pltpu.VMEM((9504, 256), jnp.bfloat16)],
        compiler_params=pltpu.CompilerParams(
            dimension_semantics=("parallel", "parallel"),
            vmem_limit_bytes=40 * 1024 * 1024),
    )(xs, *k1_w, *k1_sb)

    b1r = b1p.reshape(8, 9216, 2, 128).transpose(0, 2, 1, 3).reshape(16, 9216, 128)

    k2_w = [_gw(conv3_w, 2), _gw(conv4_w, 2), _gw(conv5_w, 2)]
    k2_sb = [jnp.tile(s, (1, 2)) for s in (conv3_sb, conv4_sb, conv5_sb)]
    c2g = lambda i: (0, 0)
    c3g = lambda i: (0, 0, 0)

    b2p = pl.pallas_call(
        _k2_kernel,
        out_shape=jax.ShapeDtypeStruct((16, 900, 256), jnp.bfloat16),
        grid=(16,),
        in_specs=[pl.BlockSpec((1, 9216, 128), lambda i: (i, 0, 0))]
        + [pl.BlockSpec(a.shape, c3g) for a in k2_w]
        + [pl.BlockSpec(a.shape, c2g) for a in k2_sb],
        out_specs=pl.BlockSpec((1, 900, 256), lambda i: (i, 0, 0)),
        scratch_shapes=[pltpu.VMEM((9024, 256), jnp.bfloat16),
                        pltpu.VMEM((8832, 256), jnp.bfloat16),
                        pltpu.VMEM((8640, 256), jnp.bfloat16)],
        compiler_params=pltpu.CompilerParams(
            dimension_semantics=("parallel",),
            vmem_limit_bytes=40 * 1024 * 1024),
    )(b1r, *k2_w, *k2_sb)

    b2r = b2p.reshape(16, 900, 2, 128).transpose(0, 2, 1, 3).reshape(32, 900, 128)

    k3_w = [_gw(w, 1) for w in (conv6_w, conv7_w, conv8_w, conv9_w, conv10_w)]
    k3_sb = [conv6_sb, conv7_sb, conv8_sb, conv9_sb, conv10_sb]
    c2 = lambda i: (0, 0)
    c3 = lambda i: (0, 0, 0)

    out = pl.pallas_call(
        _k3_kernel,
        out_shape=jax.ShapeDtypeStruct((_B, 1, 20), f32),
        grid=(_B,),
        in_specs=[pl.BlockSpec((1, 900, 128), lambda i: (i, 0, 0))]
        + [pl.BlockSpec(a.shape, c3) for a in k3_w]
        + [pl.BlockSpec(a.shape, c2) for a in k3_sb]
        + [pl.BlockSpec(w1.shape, c2), pl.BlockSpec((1, 100), c2),
           pl.BlockSpec(w2.shape, c2), pl.BlockSpec((1, 20), c2)],
        out_specs=pl.BlockSpec((1, 1, 20), lambda i: (i, 0, 0)),
        scratch_shapes=[pltpu.VMEM((848, 256), jnp.bfloat16),
                        pltpu.VMEM((784, 256), jnp.bfloat16),
                        pltpu.VMEM((32, 256), jnp.bfloat16)],
        compiler_params=pltpu.CompilerParams(
            dimension_semantics=("parallel",),
            vmem_limit_bytes=40 * 1024 * 1024),
    )(b2r, *k3_w, *k3_sb, w1, b1.reshape(1, 100), w2, b2.reshape(1, 20))
    return out.reshape(_B, 20)


# speedup vs baseline: 1.1666x; 1.0619x over previous
import jax
import jax.numpy as jnp
from jax.experimental import pallas as pl
from jax.experimental.pallas import tpu as pltpu

_B = 32
_H = 198
_S0 = _H * _H


def _conv_step(src, dst, w_ref, sb_ref, stride, m, cin, cout, tm=512):
    scale = sb_ref[0:1, :]
    bias = sb_ref[1:2, :]
    for c0 in range(0, m, tm):
        sz = min(tm, m - c0)
        acc = None
        for dy in range(3):
            for dx in range(3):
                t = dy * 3 + dx
                off = c0 + dy * stride + dx
                a = src[off:off + sz, 0:cin]
                p = jnp.dot(a, w_ref[t], preferred_element_type=jnp.float32)
                acc = p if acc is None else acc + p
        dst[c0:c0 + sz, 0:cout] = jnp.maximum(
            acc * scale + bias, 0.0).astype(jnp.bfloat16)


def _pool_step(src, dst, k, stride, n_rows, n_cols, c):
    span = k * n_cols
    for r in range(n_rows):
        m1 = None
        for i in range(k):
            base = (k * r + i) * stride
            sl = src[base:base + span, 0:c]
            m1 = sl if m1 is None else jnp.maximum(m1, sl)
        if k == 2:
            w32 = pltpu.bitcast(m1, jnp.int32)
            lo = pltpu.bitcast(jnp.left_shift(w32, 16), jnp.float32)
            hi = pltpu.bitcast(jnp.bitwise_and(w32, jnp.int32(-65536)),
                               jnp.float32)
            red = jnp.maximum(lo, hi).astype(jnp.bfloat16)
        else:
            red = jnp.max(m1.reshape(n_cols, k, c), axis=1)
        dst[r * n_cols:(r + 1) * n_cols, 0:c] = red


def _k1_kernel(x_ref, w0, w1, w2, s0, s1, s2, out_ref, sa, sb, sc):
    _conv_step(x_ref.at[0, 0], sa, w0, s0, 198, 10294, 12, 128)
    _conv_step(sa, sb, w1, s1, 198, 9896, 128, 128)
    _conv_step(sb, sc, w2, s2, 198, 9498, 128, 256)
    _pool_step(sc, out_ref.at[0, 0], 2, 198, 24, 96, 256)


def _k2_kernel(x_ref, w3, w4, w5, s3, s4, s5, out_ref, sa, sb, sc):
    _conv_step(x_ref.at[0], sa, w3, s3, 96, 9022, 128, 256)
    _conv_step(sa, sb, w4, s4, 96, 8828, 256, 256)
    _conv_step(sb, sc, w5, s5, 96, 8634, 256, 256)
    _pool_step(sc, out_ref.at[0], 3, 96, 30, 30, 256)


def _k3_kernel(x_ref, w6, w7, w8, w9, w10, s6, s7, s8, s9, s10,
               w1_ref, b1_ref, w2_ref, b2_ref, out_ref, sa, sb, sc):
    _conv_step(x_ref.at[0], sa, w6, s6, 30, 838, 128, 256)
    _conv_step(sa, sb, w7, s7, 30, 776, 256, 256)
    _conv_step(sb, sa, w8, s8, 30, 714, 256, 256)
    _pool_step(sa, sb, 3, 30, 8, 8, 256)
    _conv_step(sb, sa, w9, s9, 8, 46, 256, 256)
    _conv_step(sa, sc, w10, s10, 8, 28, 256, 256)
    s = (sc[0:4, :].astype(jnp.float32)
         + sc[8:12, :].astype(jnp.float32)
         + sc[16:20, :].astype(jnp.float32)
         + sc[24:28, :].astype(jnp.float32))
    xp = jnp.sum(s, axis=0, keepdims=True) * (1.0 / 16.0)
    h = jnp.dot(xp, w1_ref[...], preferred_element_type=jnp.float32) + b1_ref[...]
    h = jnp.maximum(h, 0.0)
    out_ref[0] = jnp.dot(h, w2_ref[...], preferred_element_type=jnp.float32) + b2_ref[...]


def _gw(w, g):
    k9 = w.reshape(9, w.shape[2], w.shape[3])
    if g == 1:
        return k9
    eye = jnp.eye(g, dtype=w.dtype)
    bd = jnp.einsum('gh,tij->tgihj', eye, k9)
    return bd.reshape(9, g * w.shape[2], g * w.shape[3])


def _const_specs(arrs):
    c2 = lambda i, j: (0, 0)
    c3 = lambda i, j: (0, 0, 0)
    return [pl.BlockSpec(a.shape, c3 if a.ndim == 3 else c2) for a in arrs]


def kernel(x, conv0_w, conv0_sb, conv1_w, conv1_sb, conv2_w, conv2_sb,
           conv3_w, conv3_sb, conv4_w, conv4_sb, conv5_w, conv5_sb,
           conv6_w, conv6_sb, conv7_w, conv7_sb, conv8_w, conv8_sb,
           conv9_w, conv9_sb, conv10_w, conv10_sb, w1, b1, w2, b2):
    f32 = jnp.float32

    xb = x.astype(jnp.bfloat16).reshape(8, 4, _S0, 3)
    xg = xb.transpose(0, 2, 1, 3).reshape(8, _S0, 12)
    xs = jnp.stack([xg[:, 198 * 48 * t: 198 * (48 * t + 54), :]
                    for t in range(4)], axis=1)

    k1_w = [_gw(conv0_w, 4), _gw(conv1_w, 4), _gw(conv2_w, 4)]
    k1_sb = [jnp.tile(s, (1, 4)) for s in (conv0_sb, conv1_sb, conv2_sb)]

    b1p = pl.pallas_call(
        _k1_kernel,
        out_shape=jax.ShapeDtypeStruct((8, 4, 2304, 256), jnp.bfloat16),
        grid=(8, 4),
        in_specs=[pl.BlockSpec((1, 1, 10692, 12), lambda i, j: (i, j, 0, 0))]
        + _const_specs(k1_w + k1_sb),
        out_specs=pl.BlockSpec((1, 1, 2304, 256), lambda i, j: (i, j, 0, 0)),
        scratch_shapes=[pltpu.VMEM((10304, 128), jnp.bfloat16),
                        pltpu.VMEM((9904, 128), jnp.bfloat16),
                        pltpu.VMEM((9504, 256), jnp.bfloat16)],
        compiler_params=pltpu.CompilerParams(
            dimension_semantics=("parallel", "parallel"),
            vmem_limit_bytes=40 * 1024 * 1024),
    )(xs, *k1_w, *k1_sb)

    b1r = b1p.reshape(8, 9216, 256)

    k2_w = [_gw(conv3_w, 2), _gw(conv4_w, 2), _gw(conv5_w, 2)]
    k2_sb = [jnp.tile(s, (1, 2)) for s in (conv3_sb, conv4_sb, conv5_sb)]
    c2g = lambda i: (0, 0)
    c3g = lambda i: (0, 0, 0)

    b2p = pl.pallas_call(
        _k2_kernel,
        out_shape=jax.ShapeDtypeStruct((16, 900, 256), jnp.bfloat16),
        grid=(16,),
        in_specs=[pl.BlockSpec((1, 9216, 128), lambda i: (i // 2, 0, i % 2))]
        + [pl.BlockSpec(a.shape, c3g) for a in k2_w]
        + [pl.BlockSpec(a.shape, c2g) for a in k2_sb],
        out_specs=pl.BlockSpec((1, 900, 256), lambda i: (i, 0, 0)),
        scratch_shapes=[pltpu.VMEM((9024, 256), jnp.bfloat16),
                        pltpu.VMEM((8832, 256), jnp.bfloat16),
                        pltpu.VMEM((8640, 256), jnp.bfloat16)],
        compiler_params=pltpu.CompilerParams(
            dimension_semantics=("parallel",),
            vmem_limit_bytes=40 * 1024 * 1024),
    )(b1r, *k2_w, *k2_sb)

    b2r = b2p

    k3_w = [_gw(w, 1) for w in (conv6_w, conv7_w, conv8_w, conv9_w, conv10_w)]
    k3_sb = [conv6_sb, conv7_sb, conv8_sb, conv9_sb, conv10_sb]
    c2 = lambda i: (0, 0)
    c3 = lambda i: (0, 0, 0)

    out = pl.pallas_call(
        _k3_kernel,
        out_shape=jax.ShapeDtypeStruct((_B, 1, 20), f32),
        grid=(_B,),
        in_specs=[pl.BlockSpec((1, 900, 128), lambda i: (i // 2, 0, i % 2))]
        + [pl.BlockSpec(a.shape, c3) for a in k3_w]
        + [pl.BlockSpec(a.shape, c2) for a in k3_sb]
        + [pl.BlockSpec(w1.shape, c2), pl.BlockSpec((1, 100), c2),
           pl.BlockSpec(w2.shape, c2), pl.BlockSpec((1, 20), c2)],
        out_specs=pl.BlockSpec((1, 1, 20), lambda i: (i, 0, 0)),
        scratch_shapes=[pltpu.VMEM((848, 256), jnp.bfloat16),
                        pltpu.VMEM((784, 256), jnp.bfloat16),
                        pltpu.VMEM((32, 256), jnp.bfloat16)],
        compiler_params=pltpu.CompilerParams(
            dimension_semantics=("parallel",),
            vmem_limit_bytes=40 * 1024 * 1024),
    )(b2r, *k3_w, *k3_sb, w1, b1.reshape(1, 100), w2, b2.reshape(1, 20))
    return out.reshape(_B, 20)
